# baseline (speedup 1.0000x reference)
import functools

import jax
import jax.numpy as jnp
from jax import lax
from jax.experimental import pallas as pl
from jax.experimental.pallas import tpu as pltpu
from jax.experimental.pallas import tpu_sc as plsc

_NUM_TABLE = 8
_TABLE_SIZE = 64
_CODE_LEN = 6
_P = 4
_TP = _NUM_TABLE * _P
_ROWS = _NUM_TABLE * _TABLE_SIZE

_B, _H, _S, _D = 2, 16, 2048, 64
_BH = _B * _H
_NCH = 8
_CH = _S // _NCH
_L = 16


def _codes_rows(sT):
    T = _NUM_TABLE
    N = sT.shape[-1]
    planes = [sT[8 * c:8 * (c + 1), :] for c in range(_CODE_LEN)]
    absp = [jnp.abs(p) for p in planes]
    base = jnp.zeros((T, N), jnp.int32)
    for c in range(_CODE_LEN):
        base = base + (planes[c] > 0).astype(jnp.int32) * (1 << c)
    ranks = []
    for c in range(_CODE_LEN):
        r = jnp.zeros((T, N), jnp.int32)
        for c2 in range(_CODE_LEN):
            if c2 == c:
                continue
            lt = absp[c2] < absp[c]
            if c2 < c:
                lt = lt | (absp[c2] == absp[c])
            r = r + lt.astype(jnp.int32)
        ranks.append(r)
    flips = []
    for i in range(_P - 1):
        f = jnp.zeros((T, N), jnp.int32)
        for c in range(_CODE_LEN):
            f = f + (ranks[c] == i).astype(jnp.int32) * (1 << c)
        flips.append(f)
    toff = jax.lax.broadcasted_iota(jnp.int32, (T, N), 0) * _TABLE_SIZE
    rows = [base + toff]
    for i in range(_P - 1):
        rows.append(jnp.bitwise_xor(base, flips[i]) + toff)
    return rows


def _codes_body(q_ref, k_ref, v_ref, w_ref, qr_ref, kr_ref, vt_ref):
    q = q_ref[0, 0]
    k = k_ref[0, 0]
    v = v_ref[0, 0]
    w = w_ref[0]
    dn_nt = (((1,), (1,)), ((), ()))
    sq = jax.lax.dot_general(w, q, dn_nt)
    sk = jax.lax.dot_general(w, k, dn_nt)
    qr_ref[0, 0] = jnp.concatenate(_codes_rows(sq), axis=0)
    kr_ref[0, 0] = jnp.concatenate(_codes_rows(sk), axis=0)
    vt_ref[0, 0] = v.T


_sc_mesh = plsc.VectorSubcoreMesh(core_axis_name="c", subcore_axis_name="s")


@functools.partial(
    pl.kernel,
    out_type=jax.ShapeDtypeStruct((_BH, _NCH, _D * _CH), jnp.float32),
    mesh=_sc_mesh,
    scratch_types=[
        pltpu.VMEM((_D * _ROWS,), jnp.float32),
        pltpu.VMEM((_ROWS,), jnp.float32),
        pltpu.VMEM((_D * _CH,), jnp.float32),
        pltpu.VMEM((_TP * _CH,), jnp.int32),
        pltpu.VMEM((_TP * _CH,), jnp.int32),
        pltpu.VMEM((_D * _CH,), jnp.float32),
        pltpu.VMEM((_CH,), jnp.float32),
    ],
    compiler_params=pltpu.CompilerParams(needs_layout_passes=False),
)
def _sc_tables(vt_hbm, qr_hbm, kr_hbm, out_hbm,
               tab, cnts, vbuf, qbuf, kbuf, obuf, cbuf):
    bh = lax.axis_index("c") * 16 + lax.axis_index("s")
    zero16 = jnp.zeros((_L,), jnp.float32)
    ones16 = jnp.ones((_L,), jnp.float32)

    def _zero(ref, n):
        def zb(j, _):
            ref[pl.ds(j * _L, _L)] = zero16
            return 0
        lax.fori_loop(0, n // _L, zb, 0, unroll=4)

    _zero(tab, _D * _ROWS)
    _zero(cnts, _ROWS)

    def build_chunk(ch, _):
        pltpu.sync_copy(vt_hbm.at[bh, ch], vbuf)
        pltpu.sync_copy(qr_hbm.at[bh, ch], qbuf)

        def g_loop(g, _):
            gbase = g * _L

            def tp_loop(tp, _):
                ridx = qbuf[pl.ds(tp * _CH + gbase, _L)]
                plsc.addupdate_scatter(cnts, [ridx], ones16)

                def d_loop(d, idx):
                    vals = vbuf[pl.ds(d * _CH + gbase, _L)]
                    plsc.addupdate_scatter(tab, [idx], vals)
                    return idx + _ROWS

                lax.fori_loop(0, _D, d_loop, ridx, unroll=8)
                return 0

            return lax.fori_loop(0, _TP, tp_loop, 0)

        return lax.fori_loop(0, _CH // _L, g_loop, 0)

    lax.fori_loop(0, _NCH, build_chunk, 0)

    def query_chunk(ch, _):
        pltpu.sync_copy(kr_hbm.at[bh, ch], kbuf)
        _zero(obuf, _D * _CH)
        _zero(cbuf, _CH)

        def g_loop(g, _):
            gbase = g * _L

            def tp_loop(tp, _):
                ridx = kbuf[pl.ds(tp * _CH + gbase, _L)]
                cb = plsc.load_gather(cnts, [ridx])
                plsc.addupdate(cbuf.at[pl.ds(gbase, _L)], cb)

                def d_loop(d, idx):
                    vals = plsc.load_gather(tab, [idx])
                    plsc.addupdate(obuf.at[pl.ds(d * _CH + gbase, _L)], vals)
                    return idx + _ROWS

                lax.fori_loop(0, _D, d_loop, ridx, unroll=8)
                return 0

            return lax.fori_loop(0, _TP, tp_loop, 0)

        lax.fori_loop(0, _CH // _L, g_loop, 0)

        def mean_g(g, _):
            gbase = g * _L
            c = cbuf[pl.ds(gbase, _L)]
            r = 1.0 / jnp.maximum(c, 1.0)

            def mean_d(d, _):
                off = d * _CH + gbase
                obuf[pl.ds(off, _L)] = obuf[pl.ds(off, _L)] * r
                return 0

            return lax.fori_loop(0, _D, mean_d, 0, unroll=4)

        lax.fori_loop(0, _CH // _L, mean_g, 0)
        pltpu.sync_copy(obuf, out_hbm.at[bh, ch])
        return 0

    lax.fori_loop(0, _NCH, query_chunk, 0)


def kernel(query_layer, key_layer, value_layer, attention_mask, projections):
    B, H, S, D = query_layer.shape
    v = value_layer * attention_mask[:, None, :, None]
    w6 = projections.transpose(0, 2, 1, 3).reshape(H, _CODE_LEN * _NUM_TABLE, D)
    qr, kr, vt = pl.pallas_call(
        _codes_body,
        grid=(B, H),
        in_specs=[
            pl.BlockSpec((1, 1, S, D), lambda b, h: (b, h, 0, 0)),
            pl.BlockSpec((1, 1, S, D), lambda b, h: (b, h, 0, 0)),
            pl.BlockSpec((1, 1, S, D), lambda b, h: (b, h, 0, 0)),
            pl.BlockSpec((1, _CODE_LEN * _NUM_TABLE, D), lambda b, h: (h, 0, 0)),
        ],
        out_specs=[
            pl.BlockSpec((1, 1, _TP, S), lambda b, h: (b, h, 0, 0)),
            pl.BlockSpec((1, 1, _TP, S), lambda b, h: (b, h, 0, 0)),
            pl.BlockSpec((1, 1, D, S), lambda b, h: (b, h, 0, 0)),
        ],
        out_shape=[
            jax.ShapeDtypeStruct((B, H, _TP, S), jnp.int32),
            jax.ShapeDtypeStruct((B, H, _TP, S), jnp.int32),
            jax.ShapeDtypeStruct((B, H, D, S), jnp.float32),
        ],
    )(query_layer, key_layer, v, w6)
    BH = B * H
    qr4 = qr.reshape(BH, _TP, _NCH, _CH).transpose(0, 2, 1, 3).reshape(BH, _NCH, _TP * _CH)
    kr4 = kr.reshape(BH, _TP, _NCH, _CH).transpose(0, 2, 1, 3).reshape(BH, _NCH, _TP * _CH)
    vt4 = vt.reshape(BH, D, _NCH, _CH).transpose(0, 2, 1, 3).reshape(BH, _NCH, D * _CH)
    out4 = _sc_tables(vt4, qr4, kr4)
    out = out4.reshape(BH, _NCH, D, _CH).transpose(0, 1, 3, 2).reshape(B, H, S, D)
    return out


# speedup vs baseline: 2.6585x; 2.6585x over previous
import functools

import jax
import jax.numpy as jnp
from jax import lax
from jax.experimental import pallas as pl
from jax.experimental.pallas import tpu as pltpu
from jax.experimental.pallas import tpu_sc as plsc

_NUM_TABLE = 8
_TABLE_SIZE = 64
_CODE_LEN = 6
_P = 4
_TP = _NUM_TABLE * _P
_ROWS = _NUM_TABLE * _TABLE_SIZE

_B, _H, _S, _D = 2, 16, 2048, 64
_BH = _B * _H
_NCH = 8
_CH = _S // _NCH
_L = 16


def _codes_rows(sT):
    T = _NUM_TABLE
    N = sT.shape[-1]
    planes = [sT[8 * c:8 * (c + 1), :] for c in range(_CODE_LEN)]
    absp = [jnp.abs(p) for p in planes]
    base = jnp.zeros((T, N), jnp.int32)
    for c in range(_CODE_LEN):
        base = base + (planes[c] > 0).astype(jnp.int32) * (1 << c)
    ranks = []
    for c in range(_CODE_LEN):
        r = jnp.zeros((T, N), jnp.int32)
        for c2 in range(_CODE_LEN):
            if c2 == c:
                continue
            lt = absp[c2] < absp[c]
            if c2 < c:
                lt = lt | (absp[c2] == absp[c])
            r = r + lt.astype(jnp.int32)
        ranks.append(r)
    flips = []
    for i in range(_P - 1):
        f = jnp.zeros((T, N), jnp.int32)
        for c in range(_CODE_LEN):
            f = f + (ranks[c] == i).astype(jnp.int32) * (1 << c)
        flips.append(f)
    toff = jax.lax.broadcasted_iota(jnp.int32, (T, N), 0) * _TABLE_SIZE
    rows = [base + toff]
    for i in range(_P - 1):
        rows.append(jnp.bitwise_xor(base, flips[i]) + toff)
    return rows


def _codes_body(q_ref, k_ref, v_ref, w_ref, qr_ref, kr_ref, vt_ref):
    q = q_ref[0, 0]
    k = k_ref[0, 0]
    v = v_ref[0, 0]
    w = w_ref[0]
    dn_nt = (((1,), (1,)), ((), ()))
    sq = jax.lax.dot_general(w, q, dn_nt)
    sk = jax.lax.dot_general(w, k, dn_nt)
    qr_ref[0, 0] = jnp.concatenate(_codes_rows(sq), axis=0)
    kr_ref[0, 0] = jnp.concatenate(_codes_rows(sk), axis=0)
    vt_ref[0, 0] = v.T


_sc_mesh = plsc.VectorSubcoreMesh(core_axis_name="c", subcore_axis_name="s")


@functools.partial(
    pl.kernel,
    out_type=jax.ShapeDtypeStruct((_BH, _NCH, _D * _CH), jnp.float32),
    mesh=_sc_mesh,
    scratch_types=[
        pltpu.VMEM((_D * _ROWS,), jnp.float32),
        pltpu.VMEM((_ROWS,), jnp.float32),
        pltpu.VMEM((_D * _CH,), jnp.float32),
        pltpu.VMEM((_TP * _CH,), jnp.int32),
        pltpu.VMEM((_TP * _CH,), jnp.int32),
        pltpu.VMEM((_D * _CH,), jnp.float32),
    ],
    compiler_params=pltpu.CompilerParams(needs_layout_passes=False),
)
def _sc_tables(vt_hbm, qr_hbm, kr_hbm, out_hbm,
               tab, cnts, vbuf, qbuf, kbuf, obuf):
    bh = lax.axis_index("c") * 16 + lax.axis_index("s")
    zero16 = jnp.zeros((_L,), jnp.float32)
    ones16 = jnp.ones((_L,), jnp.float32)

    def _zero(ref, n):
        def zb(j, _):
            ref[pl.ds(j * _L, _L)] = zero16
            return 0
        lax.fori_loop(0, n // _L, zb, 0, unroll=4)

    _zero(tab, _D * _ROWS)
    _zero(cnts, _ROWS)

    def build_chunk(ch, _):
        pltpu.sync_copy(vt_hbm.at[bh, ch], vbuf)
        pltpu.sync_copy(qr_hbm.at[bh, ch], qbuf)

        def g_loop(g, _):
            gbase = g * _L
            ridx = [qbuf[pl.ds(tp * _CH + gbase, _L)] for tp in range(_TP)]
            for tp in range(_TP):
                plsc.addupdate_scatter(cnts, [ridx[tp]], ones16)

            def d_loop(d, idxs):
                vals = vbuf[pl.ds(d * _CH + gbase, _L)]
                new = []
                for tp in range(_TP):
                    plsc.addupdate_scatter(tab, [idxs[tp]], vals)
                    new.append(idxs[tp] + _ROWS)
                return tuple(new)

            lax.fori_loop(0, _D, d_loop, tuple(ridx), unroll=2)
            return 0

        return lax.fori_loop(0, _CH // _L, g_loop, 0)

    lax.fori_loop(0, _NCH, build_chunk, 0)

    def query_chunk(ch, _):
        pltpu.sync_copy(kr_hbm.at[bh, ch], kbuf)

        def g_loop(g, _):
            gbase = g * _L
            ridx = [kbuf[pl.ds(tp * _CH + gbase, _L)] for tp in range(_TP)]
            cacc = zero16
            for tp in range(_TP):
                cacc = cacc + plsc.load_gather(cnts, [ridx[tp]])
            recip = 1.0 / jnp.maximum(cacc, 1.0)

            def d_loop(d, idxs):
                acc = None
                new = []
                for tp in range(_TP):
                    val = plsc.load_gather(tab, [idxs[tp]])
                    acc = val if acc is None else acc + val
                    new.append(idxs[tp] + _ROWS)
                obuf[pl.ds(d * _CH + gbase, _L)] = acc * recip
                return tuple(new)

            lax.fori_loop(0, _D, d_loop, tuple(ridx), unroll=2)
            return 0

        lax.fori_loop(0, _CH // _L, g_loop, 0)
        pltpu.sync_copy(obuf, out_hbm.at[bh, ch])
        return 0

    lax.fori_loop(0, _NCH, query_chunk, 0)


def kernel(query_layer, key_layer, value_layer, attention_mask, projections):
    B, H, S, D = query_layer.shape
    v = value_layer * attention_mask[:, None, :, None]
    w6 = projections.transpose(0, 2, 1, 3).reshape(H, _CODE_LEN * _NUM_TABLE, D)
    qr, kr, vt = pl.pallas_call(
        _codes_body,
        grid=(B, H),
        in_specs=[
            pl.BlockSpec((1, 1, S, D), lambda b, h: (b, h, 0, 0)),
            pl.BlockSpec((1, 1, S, D), lambda b, h: (b, h, 0, 0)),
            pl.BlockSpec((1, 1, S, D), lambda b, h: (b, h, 0, 0)),
            pl.BlockSpec((1, _CODE_LEN * _NUM_TABLE, D), lambda b, h: (h, 0, 0)),
        ],
        out_specs=[
            pl.BlockSpec((1, 1, _TP, S), lambda b, h: (b, h, 0, 0)),
            pl.BlockSpec((1, 1, _TP, S), lambda b, h: (b, h, 0, 0)),
            pl.BlockSpec((1, 1, D, S), lambda b, h: (b, h, 0, 0)),
        ],
        out_shape=[
            jax.ShapeDtypeStruct((B, H, _TP, S), jnp.int32),
            jax.ShapeDtypeStruct((B, H, _TP, S), jnp.int32),
            jax.ShapeDtypeStruct((B, H, D, S), jnp.float32),
        ],
    )(query_layer, key_layer, v, w6)
    BH = B * H
    qr4 = qr.reshape(BH, _TP, _NCH, _CH).transpose(0, 2, 1, 3).reshape(BH, _NCH, _TP * _CH)
    kr4 = kr.reshape(BH, _TP, _NCH, _CH).transpose(0, 2, 1, 3).reshape(BH, _NCH, _TP * _CH)
    vt4 = vt.reshape(BH, D, _NCH, _CH).transpose(0, 2, 1, 3).reshape(BH, _NCH, D * _CH)
    out4 = _sc_tables(vt4, qr4, kr4)
    out = out4.reshape(BH, _NCH, D, _CH).transpose(0, 1, 3, 2).reshape(B, H, S, D)
    return out


# speedup vs baseline: 3.6645x; 1.3784x over previous
import functools

import jax
import jax.numpy as jnp
from jax import lax
from jax.experimental import pallas as pl
from jax.experimental.pallas import tpu as pltpu
from jax.experimental.pallas import tpu_sc as plsc

_NUM_TABLE = 8
_TABLE_SIZE = 64
_CODE_LEN = 6
_P = 4
_TP = _NUM_TABLE * _P
_ROWS = _NUM_TABLE * _TABLE_SIZE

_B, _H, _S, _D = 2, 16, 2048, 64
_BH = _B * _H
_NCH = 8
_CH = _S // _NCH
_L = 16


def _codes_rows(sT):
    T = _NUM_TABLE
    N = sT.shape[-1]
    planes = [sT[8 * c:8 * (c + 1), :] for c in range(_CODE_LEN)]
    absp = [jnp.abs(p) for p in planes]
    base = jnp.zeros((T, N), jnp.int32)
    for c in range(_CODE_LEN):
        base = base + (planes[c] > 0).astype(jnp.int32) * (1 << c)
    ranks = []
    for c in range(_CODE_LEN):
        r = jnp.zeros((T, N), jnp.int32)
        for c2 in range(_CODE_LEN):
            if c2 == c:
                continue
            lt = absp[c2] < absp[c]
            if c2 < c:
                lt = lt | (absp[c2] == absp[c])
            r = r + lt.astype(jnp.int32)
        ranks.append(r)
    flips = []
    for i in range(_P - 1):
        f = jnp.zeros((T, N), jnp.int32)
        for c in range(_CODE_LEN):
            f = f + (ranks[c] == i).astype(jnp.int32) * (1 << c)
        flips.append(f)
    toff = jax.lax.broadcasted_iota(jnp.int32, (T, N), 0) * _TABLE_SIZE
    rows = [base + toff]
    for i in range(_P - 1):
        rows.append(jnp.bitwise_xor(base, flips[i]) + toff)
    return rows


def _codes_body(q_ref, k_ref, w_ref, qr_ref, kr_ref):
    q = q_ref[0, 0]
    k = k_ref[0, 0]
    w = w_ref[0]
    dn_nt = (((1,), (1,)), ((), ()))
    sq = jax.lax.dot_general(w, q, dn_nt)
    sk = jax.lax.dot_general(w, k, dn_nt)
    qr_ref[0, 0] = jnp.concatenate(_codes_rows(sq), axis=0)
    kr_ref[0, 0] = jnp.concatenate(_codes_rows(sk), axis=0)


_sc_mesh = plsc.VectorSubcoreMesh(core_axis_name="c", subcore_axis_name="s")


@functools.partial(
    pl.kernel,
    out_type=jax.ShapeDtypeStruct((_BH, _NCH, _CH * _D), jnp.float32),
    mesh=_sc_mesh,
    scratch_types=[
        pltpu.VMEM((_ROWS * _D,), jnp.float32),
        pltpu.VMEM((_ROWS,), jnp.float32),
        pltpu.VMEM((_CH * _D,), jnp.float32),
        pltpu.VMEM((_CH * _TP,), jnp.int32),
        pltpu.VMEM((_TP * _CH,), jnp.int32),
        pltpu.VMEM((_CH * _D,), jnp.float32),
        pltpu.VMEM((_CH + _L,), jnp.float32),
    ],
    compiler_params=pltpu.CompilerParams(needs_layout_passes=False),
)
def _sc_tables(v_hbm, qrn_hbm, qrt_hbm, krn_hbm, krt_hbm, out_hbm,
               tab, cnts, vbuf, nbuf, tpbuf, obuf, rbuf):
    bh = lax.axis_index("c") * 16 + lax.axis_index("s")
    zero16 = jnp.zeros((_L,), jnp.float32)
    ones16 = jnp.ones((_L,), jnp.float32)
    nsub = _D // _L

    def _zero(ref, n):
        def zb(j, _):
            ref[pl.ds(j * _L, _L)] = zero16
            return 0
        lax.fori_loop(0, n // _L, zb, 0, unroll=4)

    _zero(tab, _ROWS * _D)
    _zero(cnts, _ROWS)

    def build_chunk(ch, _):
        pltpu.sync_copy(v_hbm.at[bh, ch], vbuf)
        pltpu.sync_copy(qrn_hbm.at[bh, ch], nbuf)
        pltpu.sync_copy(qrt_hbm.at[bh, ch], tpbuf)

        def cnt_loop(g, _):
            for tp in range(_TP):
                ridx = tpbuf[pl.ds(tp * _CH + g * _L, _L)]
                plsc.addupdate_scatter(cnts, [ridx], ones16)
            return 0

        lax.fori_loop(0, _CH // _L, cnt_loop, 0)

        def n_loop(n, _):
            vv = [vbuf[pl.ds(n * _D + j * _L, _L)] for j in range(nsub)]
            nb = n * _TP
            rv = [nbuf[pl.ds(nb + h * _L, _L)] for h in range(_TP // _L)]
            for tp in range(_TP):
                row = rv[tp // _L][tp % _L]
                base = row * _D
                for j in range(nsub):
                    plsc.addupdate(tab.at[pl.ds(base + j * _L, _L)], vv[j])
            return 0

        return lax.fori_loop(0, _CH, n_loop, 0)

    lax.fori_loop(0, _NCH, build_chunk, 0)

    def query_chunk(ch, _):
        pltpu.sync_copy(krn_hbm.at[bh, ch], nbuf)
        pltpu.sync_copy(krt_hbm.at[bh, ch], tpbuf)

        def cnt_loop(g, _):
            cacc = zero16
            for tp in range(_TP):
                ridx = tpbuf[pl.ds(tp * _CH + g * _L, _L)]
                cacc = cacc + plsc.load_gather(cnts, [ridx])
            rbuf[pl.ds(g * _L, _L)] = 1.0 / jnp.maximum(cacc, 1.0)
            return 0

        lax.fori_loop(0, _CH // _L, cnt_loop, 0)

        def n_loop(n, _):
            acc = [zero16] * nsub
            nb = n * _TP
            rv = [nbuf[pl.ds(nb + h * _L, _L)] for h in range(_TP // _L)]
            for tp in range(_TP):
                row = rv[tp // _L][tp % _L]
                base = row * _D
                for j in range(nsub):
                    acc[j] = acc[j] + tab[pl.ds(base + j * _L, _L)]
            r = rbuf[pl.ds(n, _L)][0]
            for j in range(nsub):
                obuf[pl.ds(n * _D + j * _L, _L)] = acc[j] * r
            return 0

        lax.fori_loop(0, _CH, n_loop, 0)
        pltpu.sync_copy(obuf, out_hbm.at[bh, ch])
        return 0

    lax.fori_loop(0, _NCH, query_chunk, 0)


def kernel(query_layer, key_layer, value_layer, attention_mask, projections):
    B, H, S, D = query_layer.shape
    v = value_layer * attention_mask[:, None, :, None]
    w6 = projections.transpose(0, 2, 1, 3).reshape(H, _CODE_LEN * _NUM_TABLE, D)
    qr, kr = pl.pallas_call(
        _codes_body,
        grid=(B, H),
        in_specs=[
            pl.BlockSpec((1, 1, S, D), lambda b, h: (b, h, 0, 0)),
            pl.BlockSpec((1, 1, S, D), lambda b, h: (b, h, 0, 0)),
            pl.BlockSpec((1, _CODE_LEN * _NUM_TABLE, D), lambda b, h: (h, 0, 0)),
        ],
        out_specs=[
            pl.BlockSpec((1, 1, _TP, S), lambda b, h: (b, h, 0, 0)),
            pl.BlockSpec((1, 1, _TP, S), lambda b, h: (b, h, 0, 0)),
        ],
        out_shape=[
            jax.ShapeDtypeStruct((B, H, _TP, S), jnp.int32),
            jax.ShapeDtypeStruct((B, H, _TP, S), jnp.int32),
        ],
    )(query_layer, key_layer, w6)
    BH = B * H
    qr3 = qr.reshape(BH, _TP, _NCH, _CH)
    kr3 = kr.reshape(BH, _TP, _NCH, _CH)
    qrt = qr3.transpose(0, 2, 1, 3).reshape(BH, _NCH, _TP * _CH)
    krt = kr3.transpose(0, 2, 1, 3).reshape(BH, _NCH, _TP * _CH)
    qrn = qr3.transpose(0, 2, 3, 1).reshape(BH, _NCH, _CH * _TP)
    krn = kr3.transpose(0, 2, 3, 1).reshape(BH, _NCH, _CH * _TP)
    v3 = v.reshape(BH, _NCH, _CH * D)
    out4 = _sc_tables(v3, qrn, qrt, krn, krt)
    return out4.reshape(B, H, S, D)
